# baseline (speedup 1.0000x reference)
import functools

import jax
import jax.numpy as jnp
from jax import lax
from jax.experimental import pallas as pl
from jax.experimental.pallas import tpu as pltpu
from jax.experimental.pallas import tpu_sc as plsc

_N, _E, _D = 10000, 320000, 128
_NC, _NS = 2, 16
_NW = _NC * _NS
_EPT = _E // _NW
_CH = 64
_NSL = 4
_NCHF = _EPT // _CH
_REM = _EPT - _NCHF * _CH
_NGRP = _NCHF // _NSL
_RCH = 64
_NRC = _N // _RCH
_RTAIL = _N - _NRC * _RCH

_mesh = plsc.VectorSubcoreMesh(core_axis_name="c", subcore_axis_name="s")


@functools.partial(
    pl.kernel,
    mesh=_mesh,
    out_type=jax.ShapeDtypeStruct((_NC * _N, _D), jnp.float32),
    scratch_types=[
        pltpu.VMEM((_EPT,), jnp.int32),
        pltpu.VMEM((_NSL, _CH, _D), jnp.float32),
        pltpu.VMEM((_NSL, _CH), jnp.int32),
        pltpu.VMEM((2 * _NSL, _CH), jnp.int32),
        pltpu.VMEM((_REM,), jnp.int32),
        pltpu.VMEM((_REM,), jnp.int32),
        pltpu.VMEM((_REM, _D), jnp.float32),
        pltpu.VMEM_SHARED((_N, _D), jnp.float32),
        pltpu.SemaphoreType.DMA((_NSL,)),
        pltpu.SemaphoreType.DMA((_NSL,)),
        pltpu.SemaphoreType.DMA,
        pltpu.SemaphoreType.DMA,
    ],
)
def _sc_segment_sum(pk_hbm, x_hbm, out_hbm,
                    bulk, rows, stagS, stagD, srcR, dstR, rowsR, shared,
                    gsems, ssems, bsem, wsem):
    cid = lax.axis_index("c")
    sid = lax.axis_index("s")
    gid = cid * _NS + sid
    ebase = gid * _EPT

    pltpu.async_copy(pk_hbm.at[pl.ds(pl.multiple_of(ebase, 8), _EPT)],
                     bulk, bsem)

    zero16 = jnp.zeros((16,), jnp.float32)

    def _zrow(i, carry):
        for j in range(_D // 16):
            rows[0, i, pl.ds(j * 16, 16)] = zero16
        return carry

    lax.fori_loop(0, _RCH, _zrow, 0)

    def _zcopy(fire):
        def _k(k, carry):
            c = sid + k * _NS

            @pl.when(c < _NRC)
            def _():
                off = pl.multiple_of(c * _RCH, 8)
                cp = pltpu.make_async_copy(
                    rows.at[0], shared.at[pl.ds(off, _RCH)], wsem)
                if fire:
                    cp.start()
                else:
                    cp.wait()
            return carry

        lax.fori_loop(0, (_NRC + _NS - 1) // _NS, _k, 0)

        @pl.when(sid == 0)
        def _():
            cp = pltpu.make_async_copy(
                rows.at[0].at[pl.ds(0, _RTAIL)],
                shared.at[pl.ds(_NRC * _RCH, _RTAIL)], wsem)
            if fire:
                cp.start()
            else:
                cp.wait()

    _zcopy(True)
    _zcopy(False)

    pltpu.make_async_copy(pk_hbm.at[pl.ds(0, _EPT)], bulk, bsem).wait()
    plsc.subcore_barrier()

    def _unpack(c, k, p):
        base = c * _CH
        for j in range(_CH // 16):
            w = bulk[pl.ds(base + j * 16, 16)]
            stagS[k, pl.ds(j * 16, 16)] = w & 0xFFFF
            stagD[p * _NSL + k, pl.ds(j * 16, 16)] = w >> 16

    def _fire_gather(k):
        pltpu.async_copy(x_hbm.at[stagS.at[k]], rows.at[k], gsems.at[k])

    def _wait_gather(k):
        pltpu.make_async_copy(
            x_hbm.at[stagS.at[k]], rows.at[k], gsems.at[k]).wait()

    def _fire_scatter(k, p):
        pltpu.async_copy(rows.at[k], shared.at[stagD.at[p * _NSL + k]],
                         ssems.at[k], add=True)

    def _wait_scatter(k, p):
        pltpu.make_async_copy(
            rows.at[k], shared.at[stagD.at[p * _NSL + k]],
            ssems.at[k]).wait()

    for k in range(_NSL):
        _unpack(k, k, 0)
        _fire_gather(k)

    def _body(g, p, pn):
        for k in range(_NSL):
            _wait_gather(k)
            _fire_scatter(k, p)
        for k in range(_NSL):
            _wait_scatter(k, p)
            _unpack((g + 1) * _NSL + k, k, pn)
            _fire_gather(k)

    def _dbl(t, carry):
        _body(2 * t, 0, 1)
        _body(2 * t + 1, 1, 0)
        return carry

    lax.fori_loop(0, (_NGRP - 1) // 2, _dbl, 0)

    for k in range(_NSL):
        _wait_gather(k)
        _fire_scatter(k, 0)
    wr = bulk[pl.ds(_NCHF * _CH, _REM)]
    srcR[...] = wr & 0xFFFF
    dstR[...] = wr >> 16
    pltpu.async_copy(x_hbm.at[srcR], rowsR, bsem)
    pltpu.make_async_copy(x_hbm.at[srcR], rowsR, bsem).wait()
    pltpu.sync_copy(rowsR, shared.at[dstR], add=True)
    for k in range(_NSL):
        _wait_scatter(k, 0)

    plsc.subcore_barrier()

    obase = cid * _N
    _WCH = 128
    _WNRC = _N // _WCH
    _WTAIL = _N - _WNRC * _WCH

    def _wcopy(fire):
        def _k(k, carry):
            c = sid + k * _NS

            @pl.when(c < _WNRC)
            def _():
                off = pl.multiple_of(c * _WCH, 8)
                cp = pltpu.make_async_copy(
                    shared.at[pl.ds(off, _WCH)],
                    out_hbm.at[pl.ds(obase + off, _WCH)], wsem)
                if fire:
                    cp.start()
                else:
                    cp.wait()
            return carry

        lax.fori_loop(0, (_WNRC + _NS - 1) // _NS, _k, 0)

        @pl.when(sid == 0)
        def _():
            toff = pl.multiple_of(_WNRC * _WCH, 8)
            cp = pltpu.make_async_copy(
                shared.at[pl.ds(toff, _WTAIL)],
                out_hbm.at[pl.ds(obase + toff, _WTAIL)], wsem)
            if fire:
                cp.start()
            else:
                cp.wait()

    _wcopy(True)
    _wcopy(False)


def _mlp_body(x_ref, pp_ref, w1_ref, b1_ref, w2_ref, b2_ref, o_ref):
    rst = x_ref[...] + pp_ref[0] + pp_ref[1]
    h = jnp.dot(rst, w1_ref[...], preferred_element_type=jnp.float32)
    h = jnp.maximum(h + b1_ref[...], 0.0)
    o_ref[...] = jnp.dot(h, w2_ref[...],
                         preferred_element_type=jnp.float32) + b2_ref[...]


def kernel(x, edge_index, W1, b1, gamma, beta, bn_mean, bn_var, W2, b2):
    packed = jnp.bitwise_or(edge_index[0], jnp.left_shift(edge_index[1], 16))

    partials = _sc_segment_sum(packed, x)
    pp = partials.reshape(_NC, _N, _D)

    sbn = gamma * lax.rsqrt(bn_var + 1e-5)
    w1f = W1.T * sbn[None, :]
    b1f = ((b1 - bn_mean) * sbn + beta)[None, :]
    w2f = W2.T
    b2f = b2[None, :]

    blk = 1000
    out = pl.pallas_call(
        _mlp_body,
        grid=(_N // blk,),
        in_specs=[
            pl.BlockSpec((blk, _D), lambda i: (i, 0)),
            pl.BlockSpec((_NC, blk, _D), lambda i: (0, i, 0)),
            pl.BlockSpec((_D, _D), lambda i: (0, 0)),
            pl.BlockSpec((1, _D), lambda i: (0, 0)),
            pl.BlockSpec((_D, _D), lambda i: (0, 0)),
            pl.BlockSpec((1, _D), lambda i: (0, 0)),
        ],
        out_specs=pl.BlockSpec((blk, _D), lambda i: (i, 0)),
        out_shape=jax.ShapeDtypeStruct((_N, _D), jnp.float32),
    )(x, pp, w1f, b1f, w2f, b2f)
    return out


# speedup vs baseline: 1.0210x; 1.0210x over previous
import functools

import jax
import jax.numpy as jnp
from jax import lax
from jax.experimental import pallas as pl
from jax.experimental.pallas import tpu as pltpu
from jax.experimental.pallas import tpu_sc as plsc

_N, _E, _D = 10000, 320000, 128
_NC, _NS = 2, 16
_NW = _NC * _NS
_EPT = _E // _NW
_CH = 32
_NSL = 8
_NCHF = _EPT // _CH
_REM = _EPT - _NCHF * _CH
_NGRP = _NCHF // _NSL
_RCH = 32
_NRC = _N // _RCH
_RTAIL = _N - _NRC * _RCH

_mesh = plsc.VectorSubcoreMesh(core_axis_name="c", subcore_axis_name="s")


@functools.partial(
    pl.kernel,
    mesh=_mesh,
    out_type=jax.ShapeDtypeStruct((_NC * _N, _D), jnp.float32),
    scratch_types=[
        pltpu.VMEM((_EPT,), jnp.int32),
        pltpu.VMEM((_NSL, _CH, _D), jnp.float32),
        pltpu.VMEM((_NSL, _CH), jnp.int32),
        pltpu.VMEM((2 * _NSL, _CH), jnp.int32),
        pltpu.VMEM((_REM,), jnp.int32),
        pltpu.VMEM((_REM,), jnp.int32),
        pltpu.VMEM((_REM, _D), jnp.float32),
        pltpu.VMEM_SHARED((_N, _D), jnp.float32),
        pltpu.SemaphoreType.DMA((_NSL,)),
        pltpu.SemaphoreType.DMA((_NSL,)),
        pltpu.SemaphoreType.DMA,
        pltpu.SemaphoreType.DMA,
    ],
)
def _sc_segment_sum(pk_hbm, x_hbm, out_hbm,
                    bulk, rows, stagS, stagD, srcR, dstR, rowsR, shared,
                    gsems, ssems, bsem, wsem):
    cid = lax.axis_index("c")
    sid = lax.axis_index("s")
    gid = cid * _NS + sid
    ebase = gid * _EPT

    pltpu.async_copy(pk_hbm.at[pl.ds(pl.multiple_of(ebase, 8), _EPT)],
                     bulk, bsem)

    zero16 = jnp.zeros((16,), jnp.float32)

    def _zrow(i, carry):
        for j in range(_D // 16):
            rows[0, i, pl.ds(j * 16, 16)] = zero16
        return carry

    lax.fori_loop(0, _RCH, _zrow, 0)

    def _zcopy(fire):
        def _k(k, carry):
            c = sid + k * _NS

            @pl.when(c < _NRC)
            def _():
                off = pl.multiple_of(c * _RCH, 8)
                cp = pltpu.make_async_copy(
                    rows.at[0], shared.at[pl.ds(off, _RCH)], wsem)
                if fire:
                    cp.start()
                else:
                    cp.wait()
            return carry

        lax.fori_loop(0, (_NRC + _NS - 1) // _NS, _k, 0)

        @pl.when(sid == 0)
        def _():
            cp = pltpu.make_async_copy(
                rows.at[0].at[pl.ds(0, _RTAIL)],
                shared.at[pl.ds(_NRC * _RCH, _RTAIL)], wsem)
            if fire:
                cp.start()
            else:
                cp.wait()

    _zcopy(True)
    _zcopy(False)

    pltpu.make_async_copy(pk_hbm.at[pl.ds(0, _EPT)], bulk, bsem).wait()
    plsc.subcore_barrier()

    def _unpack(c, k, p):
        base = c * _CH
        for j in range(_CH // 16):
            w = bulk[pl.ds(base + j * 16, 16)]
            stagS[k, pl.ds(j * 16, 16)] = w & 0xFFFF
            stagD[p * _NSL + k, pl.ds(j * 16, 16)] = w >> 16

    def _fire_gather(k):
        pltpu.async_copy(x_hbm.at[stagS.at[k]], rows.at[k], gsems.at[k])

    def _wait_gather(k):
        pltpu.make_async_copy(
            x_hbm.at[stagS.at[k]], rows.at[k], gsems.at[k]).wait()

    def _fire_scatter(k, p):
        pltpu.async_copy(rows.at[k], shared.at[stagD.at[p * _NSL + k]],
                         ssems.at[k], add=True)

    def _wait_scatter(k, p):
        pltpu.make_async_copy(
            rows.at[k], shared.at[stagD.at[p * _NSL + k]],
            ssems.at[k]).wait()

    for k in range(_NSL):
        _unpack(k, k, 0)
        _fire_gather(k)

    def _body(g, p, pn):
        for k in range(_NSL):
            _wait_gather(k)
            _fire_scatter(k, p)
        for k in range(_NSL):
            _wait_scatter(k, p)
            _unpack((g + 1) * _NSL + k, k, pn)
            _fire_gather(k)

    def _dbl(t, carry):
        _body(2 * t, 0, 1)
        _body(2 * t + 1, 1, 0)
        return carry

    lax.fori_loop(0, (_NGRP - 1) // 2, _dbl, 0)

    for k in range(_NSL):
        _wait_gather(k)
        _fire_scatter(k, 0)
    wr = bulk[pl.ds(_NCHF * _CH, _REM)]
    srcR[...] = wr & 0xFFFF
    dstR[...] = wr >> 16
    pltpu.async_copy(x_hbm.at[srcR], rowsR, bsem)
    pltpu.make_async_copy(x_hbm.at[srcR], rowsR, bsem).wait()
    pltpu.sync_copy(rowsR, shared.at[dstR], add=True)
    for k in range(_NSL):
        _wait_scatter(k, 0)

    plsc.subcore_barrier()

    obase = cid * _N
    _WCH = 128
    _WNRC = _N // _WCH
    _WTAIL = _N - _WNRC * _WCH

    def _wcopy(fire):
        def _k(k, carry):
            c = sid + k * _NS

            @pl.when(c < _WNRC)
            def _():
                off = pl.multiple_of(c * _WCH, 8)
                cp = pltpu.make_async_copy(
                    shared.at[pl.ds(off, _WCH)],
                    out_hbm.at[pl.ds(obase + off, _WCH)], wsem)
                if fire:
                    cp.start()
                else:
                    cp.wait()
            return carry

        lax.fori_loop(0, (_WNRC + _NS - 1) // _NS, _k, 0)

        @pl.when(sid == 0)
        def _():
            toff = pl.multiple_of(_WNRC * _WCH, 8)
            cp = pltpu.make_async_copy(
                shared.at[pl.ds(toff, _WTAIL)],
                out_hbm.at[pl.ds(obase + toff, _WTAIL)], wsem)
            if fire:
                cp.start()
            else:
                cp.wait()

    _wcopy(True)
    _wcopy(False)


def _mlp_body(x_ref, pp_ref, w1_ref, b1_ref, w2_ref, b2_ref, o_ref):
    rst = x_ref[...] + pp_ref[0] + pp_ref[1]
    h = jnp.dot(rst, w1_ref[...], preferred_element_type=jnp.float32)
    h = jnp.maximum(h + b1_ref[...], 0.0)
    o_ref[...] = jnp.dot(h, w2_ref[...],
                         preferred_element_type=jnp.float32) + b2_ref[...]


def kernel(x, edge_index, W1, b1, gamma, beta, bn_mean, bn_var, W2, b2):
    packed = jnp.bitwise_or(edge_index[0], jnp.left_shift(edge_index[1], 16))

    partials = _sc_segment_sum(packed, x)
    pp = partials.reshape(_NC, _N, _D)

    sbn = gamma * lax.rsqrt(bn_var + 1e-5)
    w1f = W1.T * sbn[None, :]
    b1f = ((b1 - bn_mean) * sbn + beta)[None, :]
    w2f = W2.T
    b2f = b2[None, :]

    blk = 1000
    out = pl.pallas_call(
        _mlp_body,
        grid=(_N // blk,),
        in_specs=[
            pl.BlockSpec((blk, _D), lambda i: (i, 0)),
            pl.BlockSpec((_NC, blk, _D), lambda i: (0, i, 0)),
            pl.BlockSpec((_D, _D), lambda i: (0, 0)),
            pl.BlockSpec((1, _D), lambda i: (0, 0)),
            pl.BlockSpec((_D, _D), lambda i: (0, 0)),
            pl.BlockSpec((1, _D), lambda i: (0, 0)),
        ],
        out_specs=pl.BlockSpec((blk, _D), lambda i: (i, 0)),
        out_shape=jax.ShapeDtypeStruct((_N, _D), jnp.float32),
    )(x, pp, w1f, b1f, w2f, b2f)
    return out


# speedup vs baseline: 1.0322x; 1.0110x over previous
import functools

import jax
import jax.numpy as jnp
from jax import lax
from jax.experimental import pallas as pl
from jax.experimental.pallas import tpu as pltpu
from jax.experimental.pallas import tpu_sc as plsc

_N, _E, _D = 10000, 320000, 128
_NC, _NS = 2, 16
_NW = _NC * _NS
_EPT = _E // _NW
_CH = 32
_NSL = 8
_NCHF = _EPT // _CH
_REM = _EPT - _NCHF * _CH
_NGRP = _NCHF // _NSL
_RCH = 16
_NRC = _N // _RCH

_mesh = plsc.VectorSubcoreMesh(core_axis_name="c", subcore_axis_name="s")


@functools.partial(
    pl.kernel,
    mesh=_mesh,
    out_type=jax.ShapeDtypeStruct((_NC * _N, _D), jnp.float32),
    scratch_types=[
        pltpu.VMEM((_EPT,), jnp.int32),
        pltpu.VMEM((_NSL, _CH, _D), jnp.float32),
        pltpu.VMEM((_NSL, _CH), jnp.int32),
        pltpu.VMEM((2 * _NSL, _CH), jnp.int32),
        pltpu.VMEM((_REM,), jnp.int32),
        pltpu.VMEM((_REM,), jnp.int32),
        pltpu.VMEM((_REM, _D), jnp.float32),
        pltpu.VMEM((_RCH, _D), jnp.float32),
        pltpu.VMEM_SHARED((_N, _D), jnp.float32),
        pltpu.SemaphoreType.DMA((_NSL,)),
        pltpu.SemaphoreType.DMA((_NSL,)),
        pltpu.SemaphoreType.DMA,
        pltpu.SemaphoreType.DMA,
    ],
)
def _sc_segment_sum(pk_hbm, x_hbm, out_hbm,
                    bulk, rows, stagS, stagD, srcR, dstR, rowsR, zbuf, shared,
                    gsems, ssems, bsem, wsem):
    cid = lax.axis_index("c")
    sid = lax.axis_index("s")
    gid = cid * _NS + sid
    ebase = gid * _EPT

    pltpu.async_copy(pk_hbm.at[pl.ds(pl.multiple_of(ebase, 8), _EPT)],
                     bulk, bsem)

    zero16 = jnp.zeros((16,), jnp.float32)

    def _zrow(i, carry):
        for j in range(_D // 16):
            zbuf[i, pl.ds(j * 16, 16)] = zero16
        return carry

    lax.fori_loop(0, _RCH, _zrow, 0)

    _NZW = (_NRC + _NS - 1) // _NS

    def _ik(k, carry):
        c = sid + k * _NS

        @pl.when(c < _NRC)
        def _():
            off = pl.multiple_of(c * _RCH, 8)
            pltpu.async_copy(zbuf, shared.at[pl.ds(off, _RCH)], wsem)

        cw = sid + (k - 16) * _NS

        @pl.when(jnp.logical_and(k >= 16, cw < _NRC))
        def _():
            offw = pl.multiple_of(cw * _RCH, 8)
            pltpu.make_async_copy(
                zbuf, shared.at[pl.ds(offw, _RCH)], wsem).wait()
        return carry

    lax.fori_loop(0, _NZW, _ik, 0)

    def _zdrain():
        def _k(k, carry):
            c = sid + k * _NS

            @pl.when(c < _NRC)
            def _():
                off = pl.multiple_of(c * _RCH, 8)
                pltpu.make_async_copy(
                    zbuf, shared.at[pl.ds(off, _RCH)], wsem).wait()
            return carry

        lax.fori_loop(_NZW - 16, _NZW, _k, 0)

    pltpu.make_async_copy(pk_hbm.at[pl.ds(0, _EPT)], bulk, bsem).wait()

    def _unpack(c, k, p):
        base = c * _CH
        for j in range(_CH // 16):
            w = bulk[pl.ds(base + j * 16, 16)]
            stagS[k, pl.ds(j * 16, 16)] = w & 0xFFFF
            stagD[p * _NSL + k, pl.ds(j * 16, 16)] = w >> 16

    def _fire_gather(k):
        pltpu.async_copy(x_hbm.at[stagS.at[k]], rows.at[k], gsems.at[k])

    def _wait_gather(k):
        pltpu.make_async_copy(
            x_hbm.at[stagS.at[k]], rows.at[k], gsems.at[k]).wait()

    def _fire_scatter(k, p):
        pltpu.async_copy(rows.at[k], shared.at[stagD.at[p * _NSL + k]],
                         ssems.at[k], add=True)

    def _wait_scatter(k, p):
        pltpu.make_async_copy(
            rows.at[k], shared.at[stagD.at[p * _NSL + k]],
            ssems.at[k]).wait()

    for k in range(_NSL):
        _unpack(k, k, 0)
        _fire_gather(k)

    _zdrain()
    plsc.subcore_barrier()

    def _body(g, p, pn):
        for k in range(_NSL):
            _wait_gather(k)
            _fire_scatter(k, p)
        for k in range(_NSL):
            _wait_scatter(k, p)
            _unpack((g + 1) * _NSL + k, k, pn)
            _fire_gather(k)

    def _dbl(t, carry):
        _body(2 * t, 0, 1)
        _body(2 * t + 1, 1, 0)
        return carry

    lax.fori_loop(0, (_NGRP - 1) // 2, _dbl, 0)

    for k in range(_NSL):
        _wait_gather(k)
        _fire_scatter(k, 0)
    wr = bulk[pl.ds(_NCHF * _CH, _REM)]
    srcR[...] = wr & 0xFFFF
    dstR[...] = wr >> 16
    pltpu.async_copy(x_hbm.at[srcR], rowsR, bsem)
    pltpu.make_async_copy(x_hbm.at[srcR], rowsR, bsem).wait()
    pltpu.sync_copy(rowsR, shared.at[dstR], add=True)
    for k in range(_NSL):
        _wait_scatter(k, 0)

    plsc.subcore_barrier()

    obase = cid * _N
    _WCH = 128
    _WNRC = _N // _WCH
    _WTAIL = _N - _WNRC * _WCH

    def _wcopy(fire):
        def _k(k, carry):
            c = sid + k * _NS

            @pl.when(c < _WNRC)
            def _():
                off = pl.multiple_of(c * _WCH, 8)
                cp = pltpu.make_async_copy(
                    shared.at[pl.ds(off, _WCH)],
                    out_hbm.at[pl.ds(obase + off, _WCH)], wsem)
                if fire:
                    cp.start()
                else:
                    cp.wait()
            return carry

        lax.fori_loop(0, (_WNRC + _NS - 1) // _NS, _k, 0)

        @pl.when(sid == 0)
        def _():
            toff = pl.multiple_of(_WNRC * _WCH, 8)
            cp = pltpu.make_async_copy(
                shared.at[pl.ds(toff, _WTAIL)],
                out_hbm.at[pl.ds(obase + toff, _WTAIL)], wsem)
            if fire:
                cp.start()
            else:
                cp.wait()

    _wcopy(True)
    _wcopy(False)


def _mlp_body(x_ref, pp_ref, w1_ref, b1_ref, w2_ref, b2_ref, o_ref):
    rst = x_ref[...] + pp_ref[0] + pp_ref[1]
    h = jnp.dot(rst, w1_ref[...], preferred_element_type=jnp.float32)
    h = jnp.maximum(h + b1_ref[...], 0.0)
    o_ref[...] = jnp.dot(h, w2_ref[...],
                         preferred_element_type=jnp.float32) + b2_ref[...]


def kernel(x, edge_index, W1, b1, gamma, beta, bn_mean, bn_var, W2, b2):
    packed = jnp.bitwise_or(edge_index[0], jnp.left_shift(edge_index[1], 16))

    partials = _sc_segment_sum(packed, x)
    pp = partials.reshape(_NC, _N, _D)

    sbn = gamma * lax.rsqrt(bn_var + 1e-5)
    w1f = W1.T * sbn[None, :]
    b1f = ((b1 - bn_mean) * sbn + beta)[None, :]
    w2f = W2.T
    b2f = b2[None, :]

    blk = 1000
    out = pl.pallas_call(
        _mlp_body,
        grid=(_N // blk,),
        in_specs=[
            pl.BlockSpec((blk, _D), lambda i: (i, 0)),
            pl.BlockSpec((_NC, blk, _D), lambda i: (0, i, 0)),
            pl.BlockSpec((_D, _D), lambda i: (0, 0)),
            pl.BlockSpec((1, _D), lambda i: (0, 0)),
            pl.BlockSpec((_D, _D), lambda i: (0, 0)),
            pl.BlockSpec((1, _D), lambda i: (0, 0)),
        ],
        out_specs=pl.BlockSpec((blk, _D), lambda i: (i, 0)),
        out_shape=jax.ShapeDtypeStruct((_N, _D), jnp.float32),
    )(x, pp, w1f, b1f, w2f, b2f)
    return out
